# baseline (speedup 1.0000x reference)
import functools

import jax
import jax.numpy as jnp
from jax import lax
from jax.experimental import pallas as pl
from jax.experimental.pallas import tpu as pltpu
from jax.experimental.pallas import tpu_sc as plsc

N = 10000
E = 320000
D = 128
DH = D // 2

NC = 2
NS = 16
NW = NC * NS

B = 128
GH = 64
NRB = 4
NCH = 80
NCH2 = 160
E_PAD = NW * NCH * B
PAD_SPREAD = 128

N_PAD = 10240
STRIPE = N_PAD // NS
BLK = 512


def _sc_mesh():
    return plsc.VectorSubcoreMesh(core_axis_name="c", subcore_axis_name="s")


@functools.partial(
    pl.kernel,
    out_type=jax.ShapeDtypeStruct((NC, N_PAD), jnp.float32),
    mesh=_sc_mesh(),
    scratch_types=[
        pltpu.VMEM((NCH, B), jnp.int32),
        pltpu.VMEM((B,), jnp.float32),
        pltpu.VMEM((STRIPE,), jnp.float32),
        pltpu.VMEM_SHARED((N_PAD,), jnp.float32),
    ],
)
def _deg_kernel(col_hbm, deg_out, idx_v, ones_v, zer_v, deg_sh):
    c = lax.axis_index("c")
    s = lax.axis_index("s")
    wid = s * NC + c
    pltpu.sync_copy(col_hbm.at[wid], idx_v)
    ones16 = jnp.ones((16,), jnp.float32)
    zero16 = jnp.zeros((16,), jnp.float32)
    for i in range(B // 16):
        ones_v[pl.ds(i * 16, 16)] = ones16
    for i in range(STRIPE // 16):
        zer_v[pl.ds(i * 16, 16)] = zero16
    pltpu.sync_copy(zer_v, deg_sh.at[pl.ds(s * STRIPE, STRIPE)])
    plsc.subcore_barrier()

    def body(j, carry):
        pltpu.sync_copy(ones_v, deg_sh.at[idx_v.at[j]], add=True)
        return carry

    lax.fori_loop(0, NCH, body, 0)
    plsc.subcore_barrier()
    pltpu.sync_copy(deg_sh.at[pl.ds(s * STRIPE, STRIPE)],
                    deg_out.at[c, pl.ds(s * STRIPE, STRIPE)])


@functools.partial(
    pl.kernel,
    out_type=[
        jax.ShapeDtypeStruct((NC, N_PAD, DH), jnp.float32),
        jax.ShapeDtypeStruct((NC, N_PAD, DH), jnp.float32),
    ],
    mesh=_sc_mesh(),
    compiler_params=pltpu.CompilerParams(use_tc_tiling_on_sc=False),
    scratch_types=[
        pltpu.VMEM((NRB, B), jnp.int32),
        pltpu.VMEM((NRB, B), jnp.int32),
        pltpu.VMEM((2, B, DH), jnp.float32),
        pltpu.VMEM((STRIPE,), jnp.float32),
        pltpu.VMEM_SHARED((N_PAD, DH), jnp.float32),
        pltpu.SemaphoreType.DMA((4,)),
        pltpu.SemaphoreType.DMA((2,)),
        pltpu.SemaphoreType.DMA((NRB,)),
        pltpu.SemaphoreType.DMA((NRB,)),
    ],
)
def _mega_kernel(y0_hbm, invd_hbm, row_hbm, col_hbm, out_hbm, y1_hbm, rowb_v,
                 colb_v, buf_v, invd_v, z_sh, gsem, ssem, rsem, csem):
    c = lax.axis_index("c")
    s = lax.axis_index("s")

    pltpu.sync_copy(invd_hbm.at[0, pl.ds(s * STRIPE, STRIPE)], invd_v)
    stripe = pl.ds(s * STRIPE, STRIPE)
    pltpu.sync_copy(y0_hbm.at[c, stripe], z_sh.at[stripe])
    plsc.subcore_barrier()

    def eload(k, rs):
        pltpu.async_copy(row_hbm.at[s, k], rowb_v.at[rs], rsem.at[rs])
        pltpu.async_copy(col_hbm.at[s, k], colb_v.at[rs], csem.at[rs])

    def ewait(k, rs):
        pltpu.make_async_copy(row_hbm.at[s, k], rowb_v.at[rs],
                              rsem.at[rs]).wait()
        pltpu.make_async_copy(col_hbm.at[s, k], colb_v.at[rs],
                              csem.at[rs]).wait()

    def _gparts(src, rs, db, h):
        idx = rowb_v.at[rs, pl.ds(h * GH, GH)]
        dst = buf_v.at[db, pl.ds(h * GH, GH)]
        sem = gsem.at[db * 2 + h]
        return src.at[idx], dst, sem

    def gissue(src, rs, db, h):
        sr, dst, sem = _gparts(src, rs, db, h)
        pltpu.async_copy(sr, dst, sem)

    def gwait(src, rs, db, h):
        sr, dst, sem = _gparts(src, rs, db, h)
        pltpu.make_async_copy(sr, dst, sem).wait()

    def sissue(rs, db):
        pltpu.async_copy(buf_v.at[db], z_sh.at[colb_v.at[rs]],
                         ssem.at[db], add=True)

    def swait(rs, db):
        pltpu.make_async_copy(buf_v.at[db], z_sh.at[colb_v.at[rs]],
                              ssem.at[db]).wait()

    def step(src, k, j4, do_swait=True, do_next=True, do_rload=True):
        db = j4 % 2
        sissue(j4 % NRB, db)
        if do_swait:
            swait((j4 - 1) % NRB, 1 - db)
        if do_next:
            ewait(k + 1, (j4 + 1) % NRB)
            gissue(src, (j4 + 1) % NRB, 1 - db, 0)
            gissue(src, (j4 + 1) % NRB, 1 - db, 1)
        if do_rload:
            eload(k + 3, (j4 + 3) % NRB)
        if do_next:
            gwait(src, (j4 + 1) % NRB, 1 - db, 0)
            gwait(src, (j4 + 1) % NRB, 1 - db, 1)

    def hop(src):
        for k in range(3):
            eload(k, k)
        ewait(0, 0)
        gissue(src, 0, 0, 0)
        gissue(src, 0, 0, 1)
        gwait(src, 0, 0, 0)
        gwait(src, 0, 0, 1)

        for k in range(4):
            step(src, k, k, do_swait=(k >= 1))

        def body(i, carry):
            k0 = i * 4
            for j in range(4):
                step(src, k0 + j, j)
            return carry

        lax.fori_loop(1, NCH2 // 4 - 1, body, 0)

        for k in range(NCH2 - 4, NCH2):
            step(src, k, k % 4, do_next=(k + 1 < NCH2),
                 do_rload=(k + 3 < NCH2))
        swait((NCH2 - 1) % NRB, (NCH2 - 1) % 2)

    hop(y0_hbm.at[c])
    plsc.subcore_barrier()

    for kk in range(STRIPE // B):
        chunk = pl.ds(s * STRIPE + kk * B, B)
        pltpu.sync_copy(z_sh.at[chunk], buf_v.at[0])

        def rgrp(g, carry):
            nv = invd_v[pl.ds(kk * B + g * 16, 16)]
            for p in range(16):
                v = lax.gather(
                    nv, jnp.full((16, 1), p, jnp.int32),
                    lax.GatherDimensionNumbers(offset_dims=(),
                                               collapsed_slice_dims=(0,),
                                               start_index_map=(0,)),
                    slice_sizes=(1,),
                    mode=lax.GatherScatterMode.PROMISE_IN_BOUNDS)
                r = g * 16 + p
                for q in range(DH // 16):
                    sl = pl.ds(q * 16, 16)
                    buf_v[0, r, sl] = buf_v[0, r, sl] * v
            return carry

        lax.fori_loop(0, B // 16, rgrp, 0)
        pltpu.sync_copy(buf_v.at[0], y1_hbm.at[c, chunk])
        pltpu.sync_copy(buf_v.at[0], z_sh.at[chunk])
    plsc.subcore_barrier()

    hop(y1_hbm.at[c])
    plsc.subcore_barrier()
    pltpu.sync_copy(z_sh.at[stripe], out_hbm.at[c, stripe])


def _deg_block(degp_ref):
    return degp_ref[0, :] + degp_ref[1, :] + 1.0


def _scale0_body(degp_ref, x_ref, y_ref, invd_ref):
    deg = _deg_block(degp_ref)
    dis = lax.rsqrt(deg)
    y = x_ref[...] * dis[:, None]
    y_ref[0] = y[:, :DH]
    y_ref[1] = y[:, DH:]
    invd_ref[...] = (1.0 / deg)[None, :]


def _final_body(degp_ref, z_ref, w_ref, b_ref, o_ref):
    dis = lax.rsqrt(_deg_block(degp_ref))
    t = jnp.concatenate([z_ref[0], z_ref[1]], axis=1)
    o_ref[...] = jnp.dot(t, w_ref[...],
                         preferred_element_type=jnp.float32) * dis[:, None] \
        + b_ref[...]


_G = N_PAD // BLK

_degp_spec = pl.BlockSpec((NC, BLK), lambda i: (0, i))
_rows_spec = pl.BlockSpec((BLK, D), lambda i: (i, 0))
_half_spec = pl.BlockSpec((NC, BLK, DH), lambda i: (0, i, 0))


def _scale0(degp, x_pad):
    return pl.pallas_call(
        _scale0_body,
        grid=(_G,),
        in_specs=[_degp_spec, _rows_spec],
        out_specs=[_half_spec, pl.BlockSpec((1, BLK), lambda i: (0, i))],
        out_shape=[
            jax.ShapeDtypeStruct((NC, N_PAD, DH), jnp.float32),
            jax.ShapeDtypeStruct((1, N_PAD), jnp.float32),
        ],
    )(degp, x_pad)


def _final(degp, z2, w, b2):
    return pl.pallas_call(
        _final_body,
        grid=(_G,),
        in_specs=[
            _degp_spec, _half_spec,
            pl.BlockSpec((D, D), lambda i: (0, 0)),
            pl.BlockSpec((1, D), lambda i: (0, 0)),
        ],
        out_specs=_rows_spec,
        out_shape=jax.ShapeDtypeStruct((N_PAD, D), jnp.float32),
    )(degp, z2, w, b2)


def kernel(inp, edge_index, W, b):
    row = edge_index[0]
    col = edge_index[1]
    pad = jnp.arange(E_PAD - E, dtype=jnp.int32) % PAD_SPREAD + N
    rowf = jnp.concatenate([row, pad])
    colf = jnp.concatenate([col, pad])
    x_pad = jnp.pad(inp, ((0, N_PAD - N), (0, 0)))

    degp = _deg_kernel(colf.reshape(NW, NCH, B))
    y0, invd = _scale0(degp, x_pad)
    z2, _y1 = _mega_kernel(y0, invd, rowf.reshape(NS, NCH2, B),
                           colf.reshape(NS, NCH2, B))
    out_full = _final(degp, z2, W, b.reshape(1, D))
    return out_full[:N]


# speedup vs baseline: 1.4230x; 1.4230x over previous
import functools

import jax
import jax.numpy as jnp
from jax import lax
from jax.experimental import pallas as pl
from jax.experimental.pallas import tpu as pltpu
from jax.experimental.pallas import tpu_sc as plsc

N = 10000
E = 320000
D = 128

NC = 2
NS = 16
NW = NC * NS

B = 64
NSLOT = 5
NCH = 160
E_PAD = NW * NCH * B
BD = 128
NCHD = 80
TRASH = 32

N_PAD = 10112
STRIPE = N_PAD // NS
N_PAD_DEG = 10240
SDEG = N_PAD_DEG // NS
BLK = 512
_G = (N_PAD + BLK - 1) // BLK


def _sc_mesh():
    return plsc.VectorSubcoreMesh(core_axis_name="c", subcore_axis_name="s")


@functools.partial(
    pl.kernel,
    out_type=jax.ShapeDtypeStruct((NC, N_PAD_DEG), jnp.float32),
    mesh=_sc_mesh(),
    scratch_types=[
        pltpu.VMEM((NCHD, BD), jnp.int32),
        pltpu.VMEM((BD,), jnp.float32),
        pltpu.VMEM((SDEG,), jnp.float32),
        pltpu.VMEM_SHARED((N_PAD_DEG,), jnp.float32),
    ],
)
def _deg_kernel(col_hbm, deg_out, idx_v, ones_v, zer_v, deg_sh):
    c = lax.axis_index("c")
    s = lax.axis_index("s")
    wid = s * NC + c
    pltpu.sync_copy(col_hbm.at[wid], idx_v)
    ones16 = jnp.ones((16,), jnp.float32)
    zero16 = jnp.zeros((16,), jnp.float32)
    for i in range(BD // 16):
        ones_v[pl.ds(i * 16, 16)] = ones16
    for i in range(SDEG // 16):
        zer_v[pl.ds(i * 16, 16)] = zero16
    pltpu.sync_copy(zer_v, deg_sh.at[pl.ds(s * SDEG, SDEG)])
    plsc.subcore_barrier()

    def body(j, carry):
        pltpu.sync_copy(ones_v, deg_sh.at[idx_v.at[j]], add=True)
        return carry

    lax.fori_loop(0, NCHD, body, 0)
    plsc.subcore_barrier()
    pltpu.sync_copy(deg_sh.at[pl.ds(s * SDEG, SDEG)],
                    deg_out.at[c, pl.ds(s * SDEG, SDEG)])


@functools.partial(
    pl.kernel,
    out_type=jax.ShapeDtypeStruct((NC, N_PAD, D), jnp.float32),
    mesh=_sc_mesh(),
    scratch_types=[
        pltpu.VMEM((8, B), jnp.int32),
        pltpu.VMEM((8, B), jnp.int32),
        pltpu.VMEM((NSLOT, B, D), jnp.float32),
        pltpu.VMEM_SHARED((N_PAD, D), jnp.float32),
        pltpu.SemaphoreType.DMA((NSLOT,)),
        pltpu.SemaphoreType.DMA((NSLOT,)),
        pltpu.SemaphoreType.DMA((8,)),
        pltpu.SemaphoreType.DMA((8,)),
    ],
)
def _prop_kernel(y_hbm, row_hbm, col_hbm, out_hbm, rowb_v, colb_v, buf_v,
                 z_sh, gsem, ssem, rsem, csem):
    c = lax.axis_index("c")
    s = lax.axis_index("s")
    wid = s * NC + c

    zero16 = jnp.zeros((16,), jnp.float32)

    def zb(bi, carry):
        for jj in range(D // 16):
            buf_v[0, bi, pl.ds(jj * 16, 16)] = zero16
        return carry

    lax.fori_loop(0, B, zb, 0)
    for k in range(STRIPE // B):
        pltpu.sync_copy(buf_v.at[0], z_sh.at[pl.ds(s * STRIPE + k * B, B)])
    pltpu.sync_copy(buf_v.at[0, pl.ds(0, STRIPE % B)],
                    z_sh.at[pl.ds(s * STRIPE + (STRIPE // B) * B, STRIPE % B)])
    plsc.subcore_barrier()

    def eload(k):
        rs = k % 8
        pltpu.async_copy(row_hbm.at[wid, k], rowb_v.at[rs], rsem.at[rs])
        pltpu.async_copy(col_hbm.at[wid, k], colb_v.at[rs], csem.at[rs])

    def ewait(k):
        rs = k % 8
        pltpu.make_async_copy(row_hbm.at[wid, k], rowb_v.at[rs],
                              rsem.at[rs]).wait()
        pltpu.make_async_copy(col_hbm.at[wid, k], colb_v.at[rs],
                              csem.at[rs]).wait()

    def _gparts(k):
        db, rs = k % NSLOT, k % 8
        return y_hbm.at[rowb_v.at[rs]], buf_v.at[db], gsem.at[db]

    def gissue(k):
        src, dst, sem = _gparts(k)
        pltpu.async_copy(src, dst, sem)

    def gwait(k):
        src, dst, sem = _gparts(k)
        pltpu.make_async_copy(src, dst, sem).wait()

    def sissue(k):
        db, cs = k % NSLOT, k % 8
        pltpu.async_copy(buf_v.at[db], z_sh.at[colb_v.at[cs]],
                         ssem.at[db], add=True)

    def swait(k):
        db, cs = k % NSLOT, k % 8
        pltpu.make_async_copy(buf_v.at[db], z_sh.at[colb_v.at[cs]],
                              ssem.at[db]).wait()

    def step(k, do_swait=True, do_g3=True, do_load=True, do_gw=True):
        sissue(k)
        if do_swait:
            swait(k - 2)
        if do_g3:
            ewait(k + 3)
            gissue(k + 3)
        if do_load:
            eload(k + 6)
        if do_gw:
            gwait(k + 1)

    for k in range(6):
        eload(k)
    for k in range(3):
        ewait(k)
        gissue(k)
    gwait(0)
    step(0, do_swait=False)
    step(1, do_swait=False)
    step(2)

    def body(k, carry):
        step(k)
        return carry

    lax.fori_loop(3, NCH - 6, body, 0)

    step(NCH - 6, do_load=False)
    step(NCH - 5, do_load=False)
    step(NCH - 4, do_load=False)
    step(NCH - 3, do_g3=False, do_load=False)
    step(NCH - 2, do_g3=False, do_load=False)
    step(NCH - 1, do_g3=False, do_load=False, do_gw=False)
    swait(NCH - 2)
    swait(NCH - 1)
    plsc.subcore_barrier()
    pltpu.sync_copy(z_sh.at[pl.ds(s * STRIPE, STRIPE)],
                    out_hbm.at[c, pl.ds(s * STRIPE, STRIPE)])


def _deg_block(degp_ref):
    return degp_ref[0, :] + degp_ref[1, :] + 1.0


def _scale0_body(degp_ref, x_ref, y_ref):
    dis = lax.rsqrt(_deg_block(degp_ref))
    y_ref[...] = x_ref[...] * dis[:, None]


def _combine_body(degp_ref, y0_ref, p_ref, y1_ref):
    inv = 1.0 / _deg_block(degp_ref)
    z = y0_ref[...] + p_ref[0] + p_ref[1]
    y1_ref[...] = z * inv[:, None]


def _final_body(degp_ref, y1_ref, q_ref, w_ref, b_ref, o_ref):
    dis = lax.rsqrt(_deg_block(degp_ref))
    z = y1_ref[...] + q_ref[0] + q_ref[1]
    t = z * dis[:, None]
    o_ref[...] = jnp.dot(t, w_ref[...],
                         preferred_element_type=jnp.float32) + b_ref[...]


_degp_spec = pl.BlockSpec((NC, BLK), lambda i: (0, i))
_rows_spec = pl.BlockSpec((BLK, D), lambda i: (i, 0))
_pair_spec = pl.BlockSpec((NC, BLK, D), lambda i: (0, i, 0))


def _scale0(degp, x):
    return pl.pallas_call(
        _scale0_body,
        grid=(_G,),
        in_specs=[_degp_spec, _rows_spec],
        out_specs=_rows_spec,
        out_shape=jax.ShapeDtypeStruct((N_PAD, D), jnp.float32),
    )(degp, x)


def _combine(degp, y0, p):
    return pl.pallas_call(
        _combine_body,
        grid=(_G,),
        in_specs=[_degp_spec, _rows_spec, _pair_spec],
        out_specs=_rows_spec,
        out_shape=jax.ShapeDtypeStruct((N_PAD, D), jnp.float32),
    )(degp, y0, p)


def _final(degp, y1, q, w, b2):
    return pl.pallas_call(
        _final_body,
        grid=(_G,),
        in_specs=[
            _degp_spec, _rows_spec, _pair_spec,
            pl.BlockSpec((D, D), lambda i: (0, 0)),
            pl.BlockSpec((1, D), lambda i: (0, 0)),
        ],
        out_specs=_rows_spec,
        out_shape=jax.ShapeDtypeStruct((N, D), jnp.float32),
    )(degp, y1, q, w, b2)


def kernel(inp, edge_index, W, b):
    row = edge_index[0]
    col = edge_index[1]
    pidx = jnp.arange(E_PAD - E, dtype=jnp.int32)
    rowf = jnp.concatenate([row, (pidx * 1237) % N])
    colf = jnp.concatenate([col, pidx % TRASH + N])
    rowp = rowf.reshape(NW, NCH, B)
    colp = colf.reshape(NW, NCH, B)

    degp = _deg_kernel(colf.reshape(NW, NCHD, BD))

    y0 = _scale0(degp, inp)
    p = _prop_kernel(y0, rowp, colp)
    y1 = _combine(degp, y0, p)
    q = _prop_kernel(y1, rowp, colp)
    return _final(degp, y1, q, W, b.reshape(1, D))


# speedup vs baseline: 1.4735x; 1.0356x over previous
import functools

import jax
import jax.numpy as jnp
from jax import lax
from jax.experimental import pallas as pl
from jax.experimental.pallas import tpu as pltpu
from jax.experimental.pallas import tpu_sc as plsc

N = 10000
E = 320000
D = 128

NC = 2
NS = 16
NW = NC * NS

B = 64
NSLOT = 5
NCH = 160
E_PAD = NW * NCH * B
BD = 128
NCHD = 80
TRASH = 32

N_PAD = 10112
STRIPE = N_PAD // NS
N_PAD_DEG = 10240
SDEG = N_PAD_DEG // NS
BLK = 512
_G = (N_PAD + BLK - 1) // BLK


def _sc_mesh():
    return plsc.VectorSubcoreMesh(core_axis_name="c", subcore_axis_name="s")


@functools.partial(
    pl.kernel,
    out_type=jax.ShapeDtypeStruct((NC, N_PAD_DEG), jnp.float32),
    mesh=_sc_mesh(),
    scratch_types=[
        pltpu.VMEM((NCHD, BD), jnp.int32),
        pltpu.VMEM((BD,), jnp.float32),
        pltpu.VMEM((SDEG,), jnp.float32),
        pltpu.VMEM_SHARED((N_PAD_DEG,), jnp.float32),
    ],
)
def _deg_kernel(col_hbm, deg_out, idx_v, ones_v, zer_v, deg_sh):
    c = lax.axis_index("c")
    s = lax.axis_index("s")
    wid = s * NC + c
    pltpu.sync_copy(col_hbm.at[wid], idx_v)
    ones16 = jnp.ones((16,), jnp.float32)
    zero16 = jnp.zeros((16,), jnp.float32)
    for i in range(BD // 16):
        ones_v[pl.ds(i * 16, 16)] = ones16
    for i in range(SDEG // 16):
        zer_v[pl.ds(i * 16, 16)] = zero16
    pltpu.sync_copy(zer_v, deg_sh.at[pl.ds(s * SDEG, SDEG)])
    plsc.subcore_barrier()

    def body(j, carry):
        pltpu.sync_copy(ones_v, deg_sh.at[idx_v.at[j]], add=True)
        return carry

    lax.fori_loop(0, NCHD, body, 0)
    plsc.subcore_barrier()
    pltpu.sync_copy(deg_sh.at[pl.ds(s * SDEG, SDEG)],
                    deg_out.at[c, pl.ds(s * SDEG, SDEG)])


@functools.partial(
    pl.kernel,
    out_type=jax.ShapeDtypeStruct((NC, N_PAD, D), jnp.float32),
    mesh=_sc_mesh(),
    scratch_types=[
        pltpu.VMEM((8, B), jnp.int32),
        pltpu.VMEM((8, B), jnp.int32),
        pltpu.VMEM((NSLOT, B, D), jnp.float32),
        pltpu.VMEM_SHARED((N_PAD, D), jnp.float32),
        pltpu.SemaphoreType.DMA((NSLOT,)),
        pltpu.SemaphoreType.DMA((NSLOT,)),
        pltpu.SemaphoreType.DMA((8,)),
        pltpu.SemaphoreType.DMA((8,)),
    ],
)
def _prop_kernel(y_hbm, row_hbm, col_hbm, out_hbm, rowb_v, colb_v, buf_v,
                 z_sh, gsem, ssem, rsem, csem):
    c = lax.axis_index("c")
    s = lax.axis_index("s")
    wid = s * NC + c

    zero16 = jnp.zeros((16,), jnp.float32)

    def zb(bi, carry):
        for jj in range(D // 16):
            buf_v[0, bi, pl.ds(jj * 16, 16)] = zero16
        return carry

    lax.fori_loop(0, B, zb, 0)
    for k in range(STRIPE // B):
        pltpu.sync_copy(buf_v.at[0], z_sh.at[pl.ds(s * STRIPE + k * B, B)])
    pltpu.sync_copy(buf_v.at[0, pl.ds(0, STRIPE % B)],
                    z_sh.at[pl.ds(s * STRIPE + (STRIPE // B) * B, STRIPE % B)])
    plsc.subcore_barrier()

    def eload(k):
        rs = k % 8
        pltpu.async_copy(row_hbm.at[wid, k], rowb_v.at[rs], rsem.at[rs])
        pltpu.async_copy(col_hbm.at[wid, k], colb_v.at[rs], csem.at[rs])

    def ewait(k):
        rs = k % 8
        pltpu.make_async_copy(row_hbm.at[wid, k], rowb_v.at[rs],
                              rsem.at[rs]).wait()
        pltpu.make_async_copy(col_hbm.at[wid, k], colb_v.at[rs],
                              csem.at[rs]).wait()

    def _gparts(k):
        db, rs = k % NSLOT, k % 8
        return y_hbm.at[rowb_v.at[rs]], buf_v.at[db], gsem.at[db]

    def gissue(k):
        src, dst, sem = _gparts(k)
        pltpu.async_copy(src, dst, sem)

    def gwait(k):
        src, dst, sem = _gparts(k)
        pltpu.make_async_copy(src, dst, sem).wait()

    def sissue(k):
        db, cs = k % NSLOT, k % 8
        pltpu.async_copy(buf_v.at[db], z_sh.at[colb_v.at[cs]],
                         ssem.at[db], add=True)

    def swait(k):
        db, cs = k % NSLOT, k % 8
        pltpu.make_async_copy(buf_v.at[db], z_sh.at[colb_v.at[cs]],
                              ssem.at[db]).wait()

    def step(k, do_swait=True, do_g4=True, do_load=True, do_gw=True):
        sissue(k)
        if do_swait:
            swait(k - 1)
        if do_g4:
            ewait(k + 4)
            gissue(k + 4)
        if do_load:
            eload(k + 7)
        if do_gw:
            gwait(k + 1)

    for k in range(7):
        eload(k)
    for k in range(4):
        ewait(k)
        gissue(k)
    gwait(0)
    step(0, do_swait=False)
    step(1)
    step(2)

    def body(k, carry):
        step(k)
        return carry

    lax.fori_loop(3, NCH - 7, body, 0)

    step(NCH - 7, do_load=False)
    step(NCH - 6, do_load=False)
    step(NCH - 5, do_load=False)
    step(NCH - 4, do_g4=False, do_load=False)
    step(NCH - 3, do_g4=False, do_load=False)
    step(NCH - 2, do_g4=False, do_load=False)
    step(NCH - 1, do_g4=False, do_load=False, do_gw=False)
    swait(NCH - 1)
    plsc.subcore_barrier()
    pltpu.sync_copy(z_sh.at[pl.ds(s * STRIPE, STRIPE)],
                    out_hbm.at[c, pl.ds(s * STRIPE, STRIPE)])


def _deg_block(degp_ref):
    return degp_ref[0, :] + degp_ref[1, :] + 1.0


def _scale0_body(degp_ref, x_ref, y_ref):
    dis = lax.rsqrt(_deg_block(degp_ref))
    y_ref[...] = x_ref[...] * dis[:, None]


def _combine_body(degp_ref, y0_ref, p_ref, y1_ref):
    inv = 1.0 / _deg_block(degp_ref)
    z = y0_ref[...] + p_ref[0] + p_ref[1]
    y1_ref[...] = z * inv[:, None]


def _final_body(degp_ref, y1_ref, q_ref, w_ref, b_ref, o_ref):
    dis = lax.rsqrt(_deg_block(degp_ref))
    z = y1_ref[...] + q_ref[0] + q_ref[1]
    t = z * dis[:, None]
    o_ref[...] = jnp.dot(t, w_ref[...],
                         preferred_element_type=jnp.float32) + b_ref[...]


_degp_spec = pl.BlockSpec((NC, BLK), lambda i: (0, i))
_rows_spec = pl.BlockSpec((BLK, D), lambda i: (i, 0))
_pair_spec = pl.BlockSpec((NC, BLK, D), lambda i: (0, i, 0))


def _scale0(degp, x):
    return pl.pallas_call(
        _scale0_body,
        grid=(_G,),
        in_specs=[_degp_spec, _rows_spec],
        out_specs=_rows_spec,
        out_shape=jax.ShapeDtypeStruct((N_PAD, D), jnp.float32),
    )(degp, x)


def _combine(degp, y0, p):
    return pl.pallas_call(
        _combine_body,
        grid=(_G,),
        in_specs=[_degp_spec, _rows_spec, _pair_spec],
        out_specs=_rows_spec,
        out_shape=jax.ShapeDtypeStruct((N_PAD, D), jnp.float32),
    )(degp, y0, p)


def _final(degp, y1, q, w, b2):
    return pl.pallas_call(
        _final_body,
        grid=(_G,),
        in_specs=[
            _degp_spec, _rows_spec, _pair_spec,
            pl.BlockSpec((D, D), lambda i: (0, 0)),
            pl.BlockSpec((1, D), lambda i: (0, 0)),
        ],
        out_specs=_rows_spec,
        out_shape=jax.ShapeDtypeStruct((N, D), jnp.float32),
    )(degp, y1, q, w, b2)


def kernel(inp, edge_index, W, b):
    row = edge_index[0]
    col = edge_index[1]
    pidx = jnp.arange(E_PAD - E, dtype=jnp.int32)
    rowf = jnp.concatenate([row, (pidx * 1237) % N])
    colf = jnp.concatenate([col, pidx % TRASH + N])
    rowp = rowf.reshape(NW, NCH, B)
    colp = colf.reshape(NW, NCH, B)

    degp = _deg_kernel(colf.reshape(NW, NCHD, BD))

    y0 = _scale0(degp, inp)
    p = _prop_kernel(y0, rowp, colp)
    y1 = _combine(degp, y0, p)
    q = _prop_kernel(y1, rowp, colp)
    return _final(degp, y1, q, W, b.reshape(1, D))


# speedup vs baseline: 1.4840x; 1.0071x over previous
import functools

import jax
import jax.numpy as jnp
from jax import lax
from jax.experimental import pallas as pl
from jax.experimental.pallas import tpu as pltpu
from jax.experimental.pallas import tpu_sc as plsc

N = 10000
E = 320000
D = 128

NC = 2
NS = 16
NW = NC * NS

B = 64
NSLOT = 5
NCH = 160
E_PAD = NW * NCH * B
BD = 128
NCHD = 80
TRASH = 32

N_PAD = 10112
STRIPE = N_PAD // NS
N_PAD_DEG = 10240
SDEG = N_PAD_DEG // NS
BLK = 512
_G = (N_PAD + BLK - 1) // BLK


def _sc_mesh():
    return plsc.VectorSubcoreMesh(core_axis_name="c", subcore_axis_name="s")


@functools.partial(
    pl.kernel,
    out_type=jax.ShapeDtypeStruct((NC, N_PAD_DEG), jnp.float32),
    mesh=_sc_mesh(),
    scratch_types=[
        pltpu.VMEM((NCHD, BD), jnp.int32),
        pltpu.VMEM((BD,), jnp.float32),
        pltpu.VMEM((SDEG,), jnp.float32),
        pltpu.VMEM_SHARED((N_PAD_DEG,), jnp.float32),
        pltpu.SemaphoreType.DMA,
    ],
)
def _deg_kernel(col_hbm, deg_out, idx_v, ones_v, zer_v, deg_sh, dsem):
    c = lax.axis_index("c")
    s = lax.axis_index("s")
    wid = s * NC + c
    pltpu.sync_copy(col_hbm.at[wid], idx_v)
    ones16 = jnp.ones((16,), jnp.float32)
    zero16 = jnp.zeros((16,), jnp.float32)
    for i in range(BD // 16):
        ones_v[pl.ds(i * 16, 16)] = ones16
    for i in range(SDEG // 16):
        zer_v[pl.ds(i * 16, 16)] = zero16
    pltpu.sync_copy(zer_v, deg_sh.at[pl.ds(s * SDEG, SDEG)])
    plsc.subcore_barrier()

    def body(j, carry):
        pltpu.async_copy(ones_v, deg_sh.at[idx_v.at[j]], dsem, add=True)
        return carry

    lax.fori_loop(0, NCHD, body, 0)

    def drain(j, carry):
        pltpu.make_async_copy(ones_v, deg_sh.at[idx_v.at[j]], dsem).wait()
        return carry

    lax.fori_loop(0, NCHD, drain, 0)
    plsc.subcore_barrier()
    pltpu.sync_copy(deg_sh.at[pl.ds(s * SDEG, SDEG)],
                    deg_out.at[c, pl.ds(s * SDEG, SDEG)])


@functools.partial(
    pl.kernel,
    out_type=jax.ShapeDtypeStruct((NC, N_PAD, D), jnp.float32),
    mesh=_sc_mesh(),
    scratch_types=[
        pltpu.VMEM((8, B), jnp.int32),
        pltpu.VMEM((8, B), jnp.int32),
        pltpu.VMEM((NSLOT, B, D), jnp.float32),
        pltpu.VMEM_SHARED((N_PAD, D), jnp.float32),
        pltpu.SemaphoreType.DMA((NSLOT,)),
        pltpu.SemaphoreType.DMA((NSLOT,)),
        pltpu.SemaphoreType.DMA((8,)),
        pltpu.SemaphoreType.DMA((8,)),
    ],
)
def _prop_kernel(y_hbm, row_hbm, col_hbm, out_hbm, rowb_v, colb_v, buf_v,
                 z_sh, gsem, ssem, rsem, csem):
    c = lax.axis_index("c")
    s = lax.axis_index("s")
    wid = s * NC + c

    zero16 = jnp.zeros((16,), jnp.float32)

    def zb(bi, carry):
        for jj in range(D // 16):
            buf_v[0, bi, pl.ds(jj * 16, 16)] = zero16
        return carry

    lax.fori_loop(0, B, zb, 0)
    for k in range(STRIPE // B):
        pltpu.sync_copy(buf_v.at[0], z_sh.at[pl.ds(s * STRIPE + k * B, B)])
    pltpu.sync_copy(buf_v.at[0, pl.ds(0, STRIPE % B)],
                    z_sh.at[pl.ds(s * STRIPE + (STRIPE // B) * B, STRIPE % B)])
    plsc.subcore_barrier()

    def eload(k):
        rs = k % 8
        pltpu.async_copy(row_hbm.at[wid, k], rowb_v.at[rs], rsem.at[rs])
        pltpu.async_copy(col_hbm.at[wid, k], colb_v.at[rs], csem.at[rs])

    def ewait(k):
        rs = k % 8
        pltpu.make_async_copy(row_hbm.at[wid, k], rowb_v.at[rs],
                              rsem.at[rs]).wait()
        pltpu.make_async_copy(col_hbm.at[wid, k], colb_v.at[rs],
                              csem.at[rs]).wait()

    def _gparts(k):
        db, rs = k % NSLOT, k % 8
        return y_hbm.at[rowb_v.at[rs]], buf_v.at[db], gsem.at[db]

    def gissue(k):
        src, dst, sem = _gparts(k)
        pltpu.async_copy(src, dst, sem)

    def gwait(k):
        src, dst, sem = _gparts(k)
        pltpu.make_async_copy(src, dst, sem).wait()

    def sissue(k):
        db, cs = k % NSLOT, k % 8
        pltpu.async_copy(buf_v.at[db], z_sh.at[colb_v.at[cs]],
                         ssem.at[db], add=True)

    def swait(k):
        db, cs = k % NSLOT, k % 8
        pltpu.make_async_copy(buf_v.at[db], z_sh.at[colb_v.at[cs]],
                              ssem.at[db]).wait()

    def step(k, do_swait=True, do_g4=True, do_load=True, do_gw=True):
        sissue(k)
        if do_swait:
            swait(k - 1)
        if do_g4:
            ewait(k + 4)
            gissue(k + 4)
        if do_load:
            eload(k + 7)
        if do_gw:
            gwait(k + 1)

    for k in range(7):
        eload(k)
    for k in range(4):
        ewait(k)
        gissue(k)
    gwait(0)
    step(0, do_swait=False)
    step(1)
    step(2)

    def body(k, carry):
        step(k)
        return carry

    lax.fori_loop(3, NCH - 7, body, 0)

    step(NCH - 7, do_load=False)
    step(NCH - 6, do_load=False)
    step(NCH - 5, do_load=False)
    step(NCH - 4, do_g4=False, do_load=False)
    step(NCH - 3, do_g4=False, do_load=False)
    step(NCH - 2, do_g4=False, do_load=False)
    step(NCH - 1, do_g4=False, do_load=False, do_gw=False)
    swait(NCH - 1)
    plsc.subcore_barrier()
    pltpu.sync_copy(z_sh.at[pl.ds(s * STRIPE, STRIPE)],
                    out_hbm.at[c, pl.ds(s * STRIPE, STRIPE)])


def _deg_block(degp_ref):
    return degp_ref[0, :] + degp_ref[1, :] + 1.0


def _scale0_body(degp_ref, x_ref, y_ref):
    dis = lax.rsqrt(_deg_block(degp_ref))
    y_ref[...] = x_ref[...] * dis[:, None]


def _combine_body(degp_ref, y0_ref, p_ref, y1_ref):
    inv = 1.0 / _deg_block(degp_ref)
    z = y0_ref[...] + p_ref[0] + p_ref[1]
    y1_ref[...] = z * inv[:, None]


def _final_body(degp_ref, y1_ref, q_ref, w_ref, b_ref, o_ref):
    dis = lax.rsqrt(_deg_block(degp_ref))
    z = y1_ref[...] + q_ref[0] + q_ref[1]
    t = z * dis[:, None]
    o_ref[...] = jnp.dot(t, w_ref[...],
                         preferred_element_type=jnp.float32) + b_ref[...]


_degp_spec = pl.BlockSpec((NC, BLK), lambda i: (0, i))
_rows_spec = pl.BlockSpec((BLK, D), lambda i: (i, 0))
_pair_spec = pl.BlockSpec((NC, BLK, D), lambda i: (0, i, 0))


def _scale0(degp, x):
    return pl.pallas_call(
        _scale0_body,
        grid=(_G,),
        in_specs=[_degp_spec, _rows_spec],
        out_specs=_rows_spec,
        out_shape=jax.ShapeDtypeStruct((N_PAD, D), jnp.float32),
    )(degp, x)


def _combine(degp, y0, p):
    return pl.pallas_call(
        _combine_body,
        grid=(_G,),
        in_specs=[_degp_spec, _rows_spec, _pair_spec],
        out_specs=_rows_spec,
        out_shape=jax.ShapeDtypeStruct((N_PAD, D), jnp.float32),
    )(degp, y0, p)


def _final(degp, y1, q, w, b2):
    return pl.pallas_call(
        _final_body,
        grid=(_G,),
        in_specs=[
            _degp_spec, _rows_spec, _pair_spec,
            pl.BlockSpec((D, D), lambda i: (0, 0)),
            pl.BlockSpec((1, D), lambda i: (0, 0)),
        ],
        out_specs=_rows_spec,
        out_shape=jax.ShapeDtypeStruct((N, D), jnp.float32),
    )(degp, y1, q, w, b2)


def kernel(inp, edge_index, W, b):
    row = edge_index[0]
    col = edge_index[1]
    pidx = jnp.arange(E_PAD - E, dtype=jnp.int32)
    rowf = jnp.concatenate([row, (pidx * 1237) % N])
    colf = jnp.concatenate([col, pidx % TRASH + N])
    rowp = rowf.reshape(NW, NCH, B)
    colp = colf.reshape(NW, NCH, B)

    degp = _deg_kernel(colf.reshape(NW, NCHD, BD))

    y0 = _scale0(degp, inp)
    p = _prop_kernel(y0, rowp, colp)
    y1 = _combine(degp, y0, p)
    q = _prop_kernel(y1, rowp, colp)
    return _final(degp, y1, q, W, b.reshape(1, D))


# speedup vs baseline: 1.5778x; 1.0632x over previous
import functools

import jax
import numpy as np
import jax.numpy as jnp
from jax import lax
from jax.experimental import pallas as pl
from jax.experimental.pallas import tpu as pltpu
from jax.experimental.pallas import tpu_sc as plsc

N = 10000
E = 320000
D = 128

NC = 2
NS = 16
NW = NC * NS

B = 64
NSLOT = 5
NCH = 160
E_PAD = NW * NCH * B
BD = 128
NCHD = 80
TRASH = 32

N_PAD = 10112
STRIPE = N_PAD // NS
N_PAD_DEG = 10240
SDEG = N_PAD_DEG // NS
BLK = 1024
_G = (N_PAD + BLK - 1) // BLK


def _sc_mesh():
    return plsc.VectorSubcoreMesh(core_axis_name="c", subcore_axis_name="s")


@functools.partial(
    pl.kernel,
    out_type=jax.ShapeDtypeStruct((NC, N_PAD_DEG), jnp.float32),
    mesh=_sc_mesh(),
    scratch_types=[
        pltpu.VMEM((NCHD, BD), jnp.int32),
        pltpu.VMEM((BD,), jnp.float32),
        pltpu.VMEM((SDEG,), jnp.float32),
        pltpu.VMEM_SHARED((N_PAD_DEG,), jnp.float32),
        pltpu.SemaphoreType.DMA,
    ],
)
def _deg_kernel(col_hbm, deg_out, idx_v, ones_v, zer_v, deg_sh, dsem):
    c = lax.axis_index("c")
    s = lax.axis_index("s")
    wid = s * NC + c
    pltpu.sync_copy(col_hbm.at[wid], idx_v)
    ones16 = jnp.ones((16,), jnp.float32)
    zero16 = jnp.zeros((16,), jnp.float32)
    for i in range(BD // 16):
        ones_v[pl.ds(i * 16, 16)] = ones16
    for i in range(SDEG // 16):
        zer_v[pl.ds(i * 16, 16)] = zero16
    pltpu.sync_copy(zer_v, deg_sh.at[pl.ds(s * SDEG, SDEG)])
    plsc.subcore_barrier()

    def body(j, carry):
        pltpu.async_copy(ones_v, deg_sh.at[idx_v.at[j]], dsem, add=True)
        return carry

    lax.fori_loop(0, NCHD, body, 0)

    def drain(j, carry):
        pltpu.make_async_copy(ones_v, deg_sh.at[idx_v.at[j]], dsem).wait()
        return carry

    lax.fori_loop(0, NCHD, drain, 0)
    plsc.subcore_barrier()
    pltpu.sync_copy(deg_sh.at[pl.ds(s * SDEG, SDEG)],
                    deg_out.at[c, pl.ds(s * SDEG, SDEG)])


@functools.partial(
    pl.kernel,
    out_type=jax.ShapeDtypeStruct((NC, N_PAD, D), jnp.float32),
    mesh=_sc_mesh(),
    scratch_types=[
        pltpu.VMEM((8, B), jnp.int32),
        pltpu.VMEM((8, B), jnp.int32),
        pltpu.VMEM((NSLOT, B, D), jnp.float32),
        pltpu.VMEM_SHARED((N_PAD, D), jnp.float32),
        pltpu.SemaphoreType.DMA((NSLOT,)),
        pltpu.SemaphoreType.DMA((NSLOT,)),
        pltpu.SemaphoreType.DMA((8,)),
        pltpu.SemaphoreType.DMA((8,)),
    ],
)
def _prop_kernel(y_hbm, row_hbm, col_hbm, out_hbm, rowb_v, colb_v, buf_v,
                 z_sh, gsem, ssem, rsem, csem):
    c = lax.axis_index("c")
    s = lax.axis_index("s")
    wid = s * NC + c

    zero16 = jnp.zeros((16,), jnp.float32)

    def zb(bi, carry):
        for jj in range(D // 16):
            buf_v[0, bi, pl.ds(jj * 16, 16)] = zero16
        return carry

    lax.fori_loop(0, B, zb, 0)
    for k in range(STRIPE // B):
        pltpu.sync_copy(buf_v.at[0], z_sh.at[pl.ds(s * STRIPE + k * B, B)])
    pltpu.sync_copy(buf_v.at[0, pl.ds(0, STRIPE % B)],
                    z_sh.at[pl.ds(s * STRIPE + (STRIPE // B) * B, STRIPE % B)])
    plsc.subcore_barrier()

    def eload(k):
        rs = k % 8
        pltpu.async_copy(row_hbm.at[wid, k], rowb_v.at[rs], rsem.at[rs])
        pltpu.async_copy(col_hbm.at[wid, k], colb_v.at[rs], csem.at[rs])

    def ewait(k):
        rs = k % 8
        pltpu.make_async_copy(row_hbm.at[wid, k], rowb_v.at[rs],
                              rsem.at[rs]).wait()
        pltpu.make_async_copy(col_hbm.at[wid, k], colb_v.at[rs],
                              csem.at[rs]).wait()

    def _gparts(k):
        db, rs = k % NSLOT, k % 8
        return y_hbm.at[rowb_v.at[rs]], buf_v.at[db], gsem.at[db]

    def gissue(k):
        src, dst, sem = _gparts(k)
        pltpu.async_copy(src, dst, sem)

    def gwait(k):
        src, dst, sem = _gparts(k)
        pltpu.make_async_copy(src, dst, sem).wait()

    def sissue(k):
        db, cs = k % NSLOT, k % 8
        pltpu.async_copy(buf_v.at[db], z_sh.at[colb_v.at[cs]],
                         ssem.at[db], add=True)

    def swait(k):
        db, cs = k % NSLOT, k % 8
        pltpu.make_async_copy(buf_v.at[db], z_sh.at[colb_v.at[cs]],
                              ssem.at[db]).wait()

    def step(k, do_swait=True, do_g4=True, do_load=True, do_gw=True):
        sissue(k)
        if do_swait:
            swait(k - 1)
        if do_g4:
            ewait(k + 4)
            gissue(k + 4)
        if do_load:
            eload(k + 7)
        if do_gw:
            gwait(k + 1)

    for k in range(7):
        eload(k)
    for k in range(4):
        ewait(k)
        gissue(k)
    gwait(0)
    step(0, do_swait=False)
    step(1)
    step(2)

    def body(k, carry):
        step(k)
        return carry

    lax.fori_loop(3, NCH - 7, body, 0)

    step(NCH - 7, do_load=False)
    step(NCH - 6, do_load=False)
    step(NCH - 5, do_load=False)
    step(NCH - 4, do_g4=False, do_load=False)
    step(NCH - 3, do_g4=False, do_load=False)
    step(NCH - 2, do_g4=False, do_load=False)
    step(NCH - 1, do_g4=False, do_load=False, do_gw=False)
    swait(NCH - 1)
    plsc.subcore_barrier()
    pltpu.sync_copy(z_sh.at[pl.ds(s * STRIPE, STRIPE)],
                    out_hbm.at[c, pl.ds(s * STRIPE, STRIPE)])


def _deg_block(degp_ref):
    return degp_ref[0, :] + degp_ref[1, :] + 1.0


def _scale0_body(degp_ref, x_ref, y_ref):
    dis = lax.rsqrt(_deg_block(degp_ref))
    y_ref[...] = x_ref[...] * dis[:, None]


def _combine_body(degp_ref, y0_ref, p_ref, y1_ref):
    inv = 1.0 / _deg_block(degp_ref)
    z = y0_ref[...] + p_ref[0] + p_ref[1]
    y1_ref[...] = z * inv[:, None]


def _final_body(degp_ref, y1_ref, q_ref, w_ref, b_ref, o_ref):
    dis = lax.rsqrt(_deg_block(degp_ref))
    z = y1_ref[...] + q_ref[0] + q_ref[1]
    t = z * dis[:, None]
    o_ref[...] = jnp.dot(t, w_ref[...],
                         preferred_element_type=jnp.float32) + b_ref[...]


_degp_spec = pl.BlockSpec((NC, BLK), lambda i: (0, i))
_rows_spec = pl.BlockSpec((BLK, D), lambda i: (i, 0))
_pair_spec = pl.BlockSpec((NC, BLK, D), lambda i: (0, i, 0))


def _scale0(degp, x):
    return pl.pallas_call(
        _scale0_body,
        grid=(_G,),
        in_specs=[_degp_spec, _rows_spec],
        out_specs=_rows_spec,
        out_shape=jax.ShapeDtypeStruct((N_PAD, D), jnp.float32),
    )(degp, x)


def _combine(degp, y0, p):
    return pl.pallas_call(
        _combine_body,
        grid=(_G,),
        in_specs=[_degp_spec, _rows_spec, _pair_spec],
        out_specs=_rows_spec,
        out_shape=jax.ShapeDtypeStruct((N_PAD, D), jnp.float32),
    )(degp, y0, p)


def _final(degp, y1, q, w, b2):
    return pl.pallas_call(
        _final_body,
        grid=(_G,),
        in_specs=[
            _degp_spec, _rows_spec, _pair_spec,
            pl.BlockSpec((D, D), lambda i: (0, 0)),
            pl.BlockSpec((1, D), lambda i: (0, 0)),
        ],
        out_specs=_rows_spec,
        out_shape=jax.ShapeDtypeStruct((N, D), jnp.float32),
    )(degp, y1, q, w, b2)


_PIDX = np.arange(E_PAD - E, dtype=np.int32)
_PAD_ROW = jnp.asarray(_PIDX * 1237 % N, dtype=jnp.int32)
_PAD_COL = jnp.asarray(_PIDX % TRASH + N, dtype=jnp.int32)

def kernel(inp, edge_index, W, b):
    row = edge_index[0]
    col = edge_index[1]
    rowf = jnp.concatenate([row, _PAD_ROW])
    colf = jnp.concatenate([col, _PAD_COL])
    rowp = rowf.reshape(NW, NCH, B)
    colp = colf.reshape(NW, NCH, B)

    degp = _deg_kernel(colf.reshape(NW, NCHD, BD))

    y0 = _scale0(degp, inp)
    p = _prop_kernel(y0, rowp, colp)
    y1 = _combine(degp, y0, p)
    q = _prop_kernel(y1, rowp, colp)
    return _final(degp, y1, q, W, b.reshape(1, D))


# speedup vs baseline: 1.6224x; 1.0283x over previous
import functools

import jax
import numpy as np
import jax.numpy as jnp
from jax import lax
from jax.experimental import pallas as pl
from jax.experimental.pallas import tpu as pltpu
from jax.experimental.pallas import tpu_sc as plsc

N = 10000
E = 320000
D = 128

NC = 2
NS = 16
NW = NC * NS

B = 64
NSLOT = 5
NCH = 160
E_PAD = NW * NCH * B
BD = 128
NCHD = 80
TRASH = 32

N_PAD = 10112
STRIPE = N_PAD // NS
N_PAD_DEG = 10240
SDEG = N_PAD_DEG // NS
BLK = 2048
_G = (N_PAD + BLK - 1) // BLK


def _sc_mesh():
    return plsc.VectorSubcoreMesh(core_axis_name="c", subcore_axis_name="s")


@functools.partial(
    pl.kernel,
    out_type=jax.ShapeDtypeStruct((NC, N_PAD_DEG), jnp.float32),
    mesh=_sc_mesh(),
    scratch_types=[
        pltpu.VMEM((NCHD, BD), jnp.int32),
        pltpu.VMEM((BD,), jnp.float32),
        pltpu.VMEM((SDEG,), jnp.float32),
        pltpu.VMEM_SHARED((N_PAD_DEG,), jnp.float32),
        pltpu.SemaphoreType.DMA,
    ],
)
def _deg_kernel(col_hbm, deg_out, idx_v, ones_v, zer_v, deg_sh, dsem):
    c = lax.axis_index("c")
    s = lax.axis_index("s")
    wid = s * NC + c
    pltpu.sync_copy(col_hbm.at[wid], idx_v)
    ones16 = jnp.ones((16,), jnp.float32)
    zero16 = jnp.zeros((16,), jnp.float32)
    for i in range(BD // 16):
        ones_v[pl.ds(i * 16, 16)] = ones16
    for i in range(SDEG // 16):
        zer_v[pl.ds(i * 16, 16)] = zero16
    pltpu.sync_copy(zer_v, deg_sh.at[pl.ds(s * SDEG, SDEG)])
    plsc.subcore_barrier()

    def body(j, carry):
        pltpu.async_copy(ones_v, deg_sh.at[idx_v.at[j]], dsem, add=True)
        return carry

    lax.fori_loop(0, NCHD, body, 0)

    def drain(j, carry):
        pltpu.make_async_copy(ones_v, deg_sh.at[idx_v.at[j]], dsem).wait()
        return carry

    lax.fori_loop(0, NCHD, drain, 0)
    plsc.subcore_barrier()
    pltpu.sync_copy(deg_sh.at[pl.ds(s * SDEG, SDEG)],
                    deg_out.at[c, pl.ds(s * SDEG, SDEG)])


@functools.partial(
    pl.kernel,
    out_type=jax.ShapeDtypeStruct((NC, N_PAD, D), jnp.float32),
    mesh=_sc_mesh(),
    scratch_types=[
        pltpu.VMEM((8, B), jnp.int32),
        pltpu.VMEM((8, B), jnp.int32),
        pltpu.VMEM((NSLOT, B, D), jnp.float32),
        pltpu.VMEM_SHARED((N_PAD, D), jnp.float32),
        pltpu.SemaphoreType.DMA((NSLOT,)),
        pltpu.SemaphoreType.DMA((NSLOT,)),
        pltpu.SemaphoreType.DMA((8,)),
        pltpu.SemaphoreType.DMA((8,)),
    ],
)
def _prop_kernel(y_hbm, row_hbm, col_hbm, out_hbm, rowb_v, colb_v, buf_v,
                 z_sh, gsem, ssem, rsem, csem):
    c = lax.axis_index("c")
    s = lax.axis_index("s")
    wid = s * NC + c

    zero16 = jnp.zeros((16,), jnp.float32)

    def zb(bi, carry):
        for jj in range(D // 16):
            buf_v[0, bi, pl.ds(jj * 16, 16)] = zero16
        return carry

    lax.fori_loop(0, B, zb, 0)
    for k in range(STRIPE // B):
        pltpu.sync_copy(buf_v.at[0], z_sh.at[pl.ds(s * STRIPE + k * B, B)])
    pltpu.sync_copy(buf_v.at[0, pl.ds(0, STRIPE % B)],
                    z_sh.at[pl.ds(s * STRIPE + (STRIPE // B) * B, STRIPE % B)])
    plsc.subcore_barrier()

    def eload(k):
        rs = k % 8
        pltpu.async_copy(row_hbm.at[wid, k], rowb_v.at[rs], rsem.at[rs])
        pltpu.async_copy(col_hbm.at[wid, k], colb_v.at[rs], csem.at[rs])

    def ewait(k):
        rs = k % 8
        pltpu.make_async_copy(row_hbm.at[wid, k], rowb_v.at[rs],
                              rsem.at[rs]).wait()
        pltpu.make_async_copy(col_hbm.at[wid, k], colb_v.at[rs],
                              csem.at[rs]).wait()

    def _gparts(k):
        db, rs = k % NSLOT, k % 8
        return y_hbm.at[rowb_v.at[rs]], buf_v.at[db], gsem.at[db]

    def gissue(k):
        src, dst, sem = _gparts(k)
        pltpu.async_copy(src, dst, sem)

    def gwait(k):
        src, dst, sem = _gparts(k)
        pltpu.make_async_copy(src, dst, sem).wait()

    def sissue(k):
        db, cs = k % NSLOT, k % 8
        pltpu.async_copy(buf_v.at[db], z_sh.at[colb_v.at[cs]],
                         ssem.at[db], add=True)

    def swait(k):
        db, cs = k % NSLOT, k % 8
        pltpu.make_async_copy(buf_v.at[db], z_sh.at[colb_v.at[cs]],
                              ssem.at[db]).wait()

    def step(k, do_swait=True, do_g4=True, do_load=True, do_gw=True):
        sissue(k)
        if do_swait:
            swait(k - 1)
        if do_g4:
            ewait(k + 4)
            gissue(k + 4)
        if do_load:
            eload(k + 7)
        if do_gw:
            gwait(k + 1)

    for k in range(7):
        eload(k)
    for k in range(4):
        ewait(k)
        gissue(k)
    gwait(0)
    step(0, do_swait=False)
    step(1)
    step(2)

    def body(k, carry):
        step(k)
        return carry

    lax.fori_loop(3, NCH - 7, body, 0)

    step(NCH - 7, do_load=False)
    step(NCH - 6, do_load=False)
    step(NCH - 5, do_load=False)
    step(NCH - 4, do_g4=False, do_load=False)
    step(NCH - 3, do_g4=False, do_load=False)
    step(NCH - 2, do_g4=False, do_load=False)
    step(NCH - 1, do_g4=False, do_load=False, do_gw=False)
    swait(NCH - 1)
    plsc.subcore_barrier()
    pltpu.sync_copy(z_sh.at[pl.ds(s * STRIPE, STRIPE)],
                    out_hbm.at[c, pl.ds(s * STRIPE, STRIPE)])


def _deg_block(degp_ref):
    return degp_ref[0, :] + degp_ref[1, :] + 1.0


def _scale0_body(degp_ref, x_ref, y_ref):
    dis = lax.rsqrt(_deg_block(degp_ref))
    y_ref[...] = x_ref[...] * dis[:, None]


def _combine_body(degp_ref, y0_ref, p_ref, y1_ref):
    inv = 1.0 / _deg_block(degp_ref)
    z = y0_ref[...] + p_ref[0] + p_ref[1]
    y1_ref[...] = z * inv[:, None]


def _final_body(degp_ref, y1_ref, q_ref, w_ref, b_ref, o_ref):
    dis = lax.rsqrt(_deg_block(degp_ref))
    z = y1_ref[...] + q_ref[0] + q_ref[1]
    t = z * dis[:, None]
    o_ref[...] = jnp.dot(t, w_ref[...],
                         preferred_element_type=jnp.float32) + b_ref[...]


_degp_spec = pl.BlockSpec((NC, BLK), lambda i: (0, i))
_rows_spec = pl.BlockSpec((BLK, D), lambda i: (i, 0))
_pair_spec = pl.BlockSpec((NC, BLK, D), lambda i: (0, i, 0))


def _scale0(degp, x):
    return pl.pallas_call(
        _scale0_body,
        grid=(_G,),
        in_specs=[_degp_spec, _rows_spec],
        out_specs=_rows_spec,
        out_shape=jax.ShapeDtypeStruct((N_PAD, D), jnp.float32),
    )(degp, x)


def _combine(degp, y0, p):
    return pl.pallas_call(
        _combine_body,
        grid=(_G,),
        in_specs=[_degp_spec, _rows_spec, _pair_spec],
        out_specs=_rows_spec,
        out_shape=jax.ShapeDtypeStruct((N_PAD, D), jnp.float32),
    )(degp, y0, p)


def _final(degp, y1, q, w, b2):
    return pl.pallas_call(
        _final_body,
        grid=(_G,),
        in_specs=[
            _degp_spec, _rows_spec, _pair_spec,
            pl.BlockSpec((D, D), lambda i: (0, 0)),
            pl.BlockSpec((1, D), lambda i: (0, 0)),
        ],
        out_specs=_rows_spec,
        out_shape=jax.ShapeDtypeStruct((N, D), jnp.float32),
    )(degp, y1, q, w, b2)


_PIDX = np.arange(E_PAD - E, dtype=np.int32)
_PAD_ROW = jnp.asarray(_PIDX * 1237 % N, dtype=jnp.int32)
_PAD_COL = jnp.asarray(_PIDX % TRASH + N, dtype=jnp.int32)

def kernel(inp, edge_index, W, b):
    row = edge_index[0]
    col = edge_index[1]
    rowf = jnp.concatenate([row, _PAD_ROW])
    colf = jnp.concatenate([col, _PAD_COL])
    rowp = rowf.reshape(NW, NCH, B)
    colp = colf.reshape(NW, NCH, B)

    degp = _deg_kernel(colf.reshape(NW, NCHD, BD))

    y0 = _scale0(degp, inp)
    p = _prop_kernel(y0, rowp, colp)
    y1 = _combine(degp, y0, p)
    q = _prop_kernel(y1, rowp, colp)
    return _final(degp, y1, q, W, b.reshape(1, D))


# speedup vs baseline: 1.6410x; 1.0115x over previous
import functools

import jax
import numpy as np
import jax.numpy as jnp
from jax import lax
from jax.experimental import pallas as pl
from jax.experimental.pallas import tpu as pltpu
from jax.experimental.pallas import tpu_sc as plsc

N = 10000
E = 320000
D = 128

NC = 2
NS = 16
NW = NC * NS

B = 64
NSLOT = 5
NCH = 160
E_PAD = NW * NCH * B
BD = 128
NCHD = 80
TRASH = 32

N_PAD = 10112
STRIPE = N_PAD // NS
N_PAD_DEG = 10240
SDEG = N_PAD_DEG // NS
BLK = 4096
_G = (N_PAD + BLK - 1) // BLK


def _sc_mesh():
    return plsc.VectorSubcoreMesh(core_axis_name="c", subcore_axis_name="s")


@functools.partial(
    pl.kernel,
    out_type=jax.ShapeDtypeStruct((NC, N_PAD_DEG), jnp.float32),
    mesh=_sc_mesh(),
    scratch_types=[
        pltpu.VMEM((NCHD, BD), jnp.int32),
        pltpu.VMEM((BD,), jnp.float32),
        pltpu.VMEM((SDEG,), jnp.float32),
        pltpu.VMEM_SHARED((N_PAD_DEG,), jnp.float32),
        pltpu.SemaphoreType.DMA,
    ],
)
def _deg_kernel(col_hbm, deg_out, idx_v, ones_v, zer_v, deg_sh, dsem):
    c = lax.axis_index("c")
    s = lax.axis_index("s")
    wid = s * NC + c
    pltpu.sync_copy(col_hbm.at[wid], idx_v)
    ones16 = jnp.ones((16,), jnp.float32)
    zero16 = jnp.zeros((16,), jnp.float32)
    for i in range(BD // 16):
        ones_v[pl.ds(i * 16, 16)] = ones16
    for i in range(SDEG // 16):
        zer_v[pl.ds(i * 16, 16)] = zero16
    pltpu.sync_copy(zer_v, deg_sh.at[pl.ds(s * SDEG, SDEG)])
    plsc.subcore_barrier()

    def body(j, carry):
        pltpu.async_copy(ones_v, deg_sh.at[idx_v.at[j]], dsem, add=True)
        return carry

    lax.fori_loop(0, NCHD, body, 0)

    def drain(j, carry):
        pltpu.make_async_copy(ones_v, deg_sh.at[idx_v.at[j]], dsem).wait()
        return carry

    lax.fori_loop(0, NCHD, drain, 0)
    plsc.subcore_barrier()
    pltpu.sync_copy(deg_sh.at[pl.ds(s * SDEG, SDEG)],
                    deg_out.at[c, pl.ds(s * SDEG, SDEG)])


@functools.partial(
    pl.kernel,
    out_type=jax.ShapeDtypeStruct((NC, N_PAD, D), jnp.float32),
    mesh=_sc_mesh(),
    scratch_types=[
        pltpu.VMEM((8, B), jnp.int32),
        pltpu.VMEM((8, B), jnp.int32),
        pltpu.VMEM((NSLOT, B, D), jnp.float32),
        pltpu.VMEM_SHARED((N_PAD, D), jnp.float32),
        pltpu.SemaphoreType.DMA((NSLOT,)),
        pltpu.SemaphoreType.DMA((NSLOT,)),
        pltpu.SemaphoreType.DMA((8,)),
        pltpu.SemaphoreType.DMA((8,)),
    ],
)
def _prop_kernel(y_hbm, row_hbm, col_hbm, out_hbm, rowb_v, colb_v, buf_v,
                 z_sh, gsem, ssem, rsem, csem):
    c = lax.axis_index("c")
    s = lax.axis_index("s")
    wid = s * NC + c

    zero16 = jnp.zeros((16,), jnp.float32)

    def zb(bi, carry):
        for jj in range(D // 16):
            buf_v[0, bi, pl.ds(jj * 16, 16)] = zero16
        return carry

    lax.fori_loop(0, B, zb, 0)
    for k in range(STRIPE // B):
        pltpu.sync_copy(buf_v.at[0], z_sh.at[pl.ds(s * STRIPE + k * B, B)])
    pltpu.sync_copy(buf_v.at[0, pl.ds(0, STRIPE % B)],
                    z_sh.at[pl.ds(s * STRIPE + (STRIPE // B) * B, STRIPE % B)])
    plsc.subcore_barrier()

    def eload(k):
        rs = k % 8
        pltpu.async_copy(row_hbm.at[wid, k], rowb_v.at[rs], rsem.at[rs])
        pltpu.async_copy(col_hbm.at[wid, k], colb_v.at[rs], csem.at[rs])

    def ewait(k):
        rs = k % 8
        pltpu.make_async_copy(row_hbm.at[wid, k], rowb_v.at[rs],
                              rsem.at[rs]).wait()
        pltpu.make_async_copy(col_hbm.at[wid, k], colb_v.at[rs],
                              csem.at[rs]).wait()

    def _gparts(k):
        db, rs = k % NSLOT, k % 8
        return y_hbm.at[rowb_v.at[rs]], buf_v.at[db], gsem.at[db]

    def gissue(k):
        src, dst, sem = _gparts(k)
        pltpu.async_copy(src, dst, sem)

    def gwait(k):
        src, dst, sem = _gparts(k)
        pltpu.make_async_copy(src, dst, sem).wait()

    def sissue(k):
        db, cs = k % NSLOT, k % 8
        pltpu.async_copy(buf_v.at[db], z_sh.at[colb_v.at[cs]],
                         ssem.at[db], add=True)

    def swait(k):
        db, cs = k % NSLOT, k % 8
        pltpu.make_async_copy(buf_v.at[db], z_sh.at[colb_v.at[cs]],
                              ssem.at[db]).wait()

    def step(k, do_swait=True, do_g4=True, do_load=True, do_gw=True):
        sissue(k)
        if do_swait:
            swait(k - 1)
        if do_g4:
            ewait(k + 4)
            gissue(k + 4)
        if do_load:
            eload(k + 7)
        if do_gw:
            gwait(k + 1)

    for k in range(7):
        eload(k)
    for k in range(4):
        ewait(k)
        gissue(k)
    gwait(0)
    step(0, do_swait=False)
    step(1)
    step(2)

    def body(k, carry):
        step(k)
        return carry

    lax.fori_loop(3, NCH - 7, body, 0)

    step(NCH - 7, do_load=False)
    step(NCH - 6, do_load=False)
    step(NCH - 5, do_load=False)
    step(NCH - 4, do_g4=False, do_load=False)
    step(NCH - 3, do_g4=False, do_load=False)
    step(NCH - 2, do_g4=False, do_load=False)
    step(NCH - 1, do_g4=False, do_load=False, do_gw=False)
    swait(NCH - 1)
    plsc.subcore_barrier()
    pltpu.sync_copy(z_sh.at[pl.ds(s * STRIPE, STRIPE)],
                    out_hbm.at[c, pl.ds(s * STRIPE, STRIPE)])


def _deg_block(degp_ref):
    return degp_ref[0, :] + degp_ref[1, :] + 1.0


def _scale0_body(degp_ref, x_ref, y_ref):
    dis = lax.rsqrt(_deg_block(degp_ref))
    y_ref[...] = x_ref[...] * dis[:, None]


def _combine_body(degp_ref, y0_ref, p_ref, y1_ref):
    inv = 1.0 / _deg_block(degp_ref)
    z = y0_ref[...] + p_ref[0] + p_ref[1]
    y1_ref[...] = z * inv[:, None]


def _final_body(degp_ref, y1_ref, q_ref, w_ref, b_ref, o_ref):
    dis = lax.rsqrt(_deg_block(degp_ref))
    z = y1_ref[...] + q_ref[0] + q_ref[1]
    t = z * dis[:, None]
    o_ref[...] = jnp.dot(t, w_ref[...],
                         preferred_element_type=jnp.float32) + b_ref[...]


_degp_spec = pl.BlockSpec((NC, BLK), lambda i: (0, i))
_rows_spec = pl.BlockSpec((BLK, D), lambda i: (i, 0))
_pair_spec = pl.BlockSpec((NC, BLK, D), lambda i: (0, i, 0))


def _scale0(degp, x):
    return pl.pallas_call(
        _scale0_body,
        grid=(_G,),
        in_specs=[_degp_spec, _rows_spec],
        out_specs=_rows_spec,
        out_shape=jax.ShapeDtypeStruct((N_PAD, D), jnp.float32),
    )(degp, x)


def _combine(degp, y0, p):
    return pl.pallas_call(
        _combine_body,
        grid=(_G,),
        in_specs=[_degp_spec, _rows_spec, _pair_spec],
        out_specs=_rows_spec,
        out_shape=jax.ShapeDtypeStruct((N_PAD, D), jnp.float32),
    )(degp, y0, p)


def _final(degp, y1, q, w, b2):
    return pl.pallas_call(
        _final_body,
        grid=(_G,),
        in_specs=[
            _degp_spec, _rows_spec, _pair_spec,
            pl.BlockSpec((D, D), lambda i: (0, 0)),
            pl.BlockSpec((1, D), lambda i: (0, 0)),
        ],
        out_specs=_rows_spec,
        out_shape=jax.ShapeDtypeStruct((N, D), jnp.float32),
    )(degp, y1, q, w, b2)


_PIDX = np.arange(E_PAD - E, dtype=np.int32)
_PAD_ROW = jnp.asarray(_PIDX * 1237 % N, dtype=jnp.int32)
_PAD_COL = jnp.asarray(_PIDX % TRASH + N, dtype=jnp.int32)

def kernel(inp, edge_index, W, b):
    row = edge_index[0]
    col = edge_index[1]
    rowf = jnp.concatenate([row, _PAD_ROW])
    colf = jnp.concatenate([col, _PAD_COL])
    rowp = rowf.reshape(NW, NCH, B)
    colp = colf.reshape(NW, NCH, B)

    degp = _deg_kernel(colf.reshape(NW, NCHD, BD))

    y0 = _scale0(degp, inp)
    p = _prop_kernel(y0, rowp, colp)
    y1 = _combine(degp, y0, p)
    q = _prop_kernel(y1, rowp, colp)
    return _final(degp, y1, q, W, b.reshape(1, D))
